# baseline (speedup 1.0000x reference)
import functools

import jax
import jax.numpy as jnp
from jax import lax
from jax.experimental import pallas as pl
from jax.experimental.pallas import tpu as pltpu
from jax.experimental.pallas import tpu_sc as plsc

B = 32
TRACKS = 16
VARS = 512
CH = 64
LANES = 16
NCORES = 2
NSUB = 16
CHUNK = 128
NCHUNK = VARS // CHUNK


@functools.partial(
    pl.kernel,
    mesh=plsc.VectorSubcoreMesh(core_axis_name="c", subcore_axis_name="s"),
    out_type=jax.ShapeDtypeStruct((B * TRACKS * VARS, CH), jnp.float32),
    scratch_types=[
        pltpu.VMEM((TRACKS, VARS), jnp.int32),
        pltpu.VMEM((NCHUNK, CHUNK), jnp.int32),
        pltpu.VMEM((VARS, CH), jnp.float32),
        pltpu.SemaphoreType.DMA,
        pltpu.SemaphoreType.DMA,
    ],
    compiler_params=pltpu.CompilerParams(use_tc_tiling_on_sc=False),
)
def _scramble_sc(x_hbm, perm_hbm, out_hbm, perm_v, idx_v, rows_v, gsem, osem):
    wid = lax.axis_index("s") * NCORES + lax.axis_index("c")
    pltpu.sync_copy(perm_hbm, perm_v)

    def per_track(t, carry):
        base = (wid * TRACKS + t) * VARS
        for j in range(NCHUNK):
            for i in range(CHUNK // LANES):
                sl = pl.ds(j * CHUNK + i * LANES, LANES)
                idx_v[j, pl.ds(i * LANES, LANES)] = perm_v[t, sl] + base
        copies = [
            pltpu.async_copy(
                x_hbm.at[idx_v.at[j]],
                rows_v.at[pl.ds(j * CHUNK, CHUNK)],
                gsem,
            )
            for j in range(NCHUNK)
        ]
        for cp in copies:
            cp.wait()
        pltpu.async_copy(rows_v, out_hbm.at[pl.ds(base, VARS)], osem).wait()
        return carry

    lax.fori_loop(0, TRACKS, per_track, 0)


def kernel(x, perm):
    x_flat = x.reshape(B * TRACKS * VARS, CH)
    perm32 = perm.astype(jnp.int32)
    out = _scramble_sc(x_flat, perm32)
    return out.reshape(B, TRACKS, VARS, CH)


# speedup vs baseline: 1.2822x; 1.2822x over previous
import functools

import jax
import jax.numpy as jnp
from jax import lax
from jax.experimental import pallas as pl
from jax.experimental.pallas import tpu as pltpu
from jax.experimental.pallas import tpu_sc as plsc

B = 32
TRACKS = 16
VARS = 512
CH = 64
NCORES = 2
VPW = VARS // 2
CK = 8
NC = VPW // CK


@functools.partial(
    pl.kernel,
    mesh=plsc.VectorSubcoreMesh(core_axis_name="c", subcore_axis_name="s"),
    out_type=jax.ShapeDtypeStruct((B, TRACKS, VARS, CH), jnp.float32),
    scratch_types=[
        pltpu.VMEM((VPW + 16,), jnp.int32),
        pltpu.VMEM((2, CK, B, CH), jnp.float32),
        pltpu.SemaphoreType.DMA,
        pltpu.SemaphoreType.DMA,
    ],
    compiler_params=pltpu.CompilerParams(needs_layout_passes=False),
)
def _scramble_sc(x_hbm, perm_hbm, out_hbm, perm_v, col_v, gsem, osem):
    wid = lax.axis_index("s") * NCORES + lax.axis_index("c")
    t = wid // 2
    v0 = (wid % 2) * VPW
    pltpu.sync_copy(perm_hbm.at[t, pl.ds(v0, VPW)], perm_v.at[pl.ds(0, VPW)])
    lanes = lax.iota(jnp.int32, 16)

    def g_desc(pv, bank, k):
        return pltpu.make_async_copy(
            x_hbm.at[:, t, pv, :], col_v.at[bank, k], gsem)

    def p_desc(j, bank, k):
        return pltpu.make_async_copy(
            col_v.at[bank, k], out_hbm.at[:, t, v0 + j, :], osem)

    def iter_c(c, carry):
        bank = lax.rem(c, 2)
        obank = 1 - bank
        base = c * CK
        pvec = perm_v[pl.ds(base, 16)]
        pvs = [jnp.sum(jnp.where(lanes == k, pvec, 0)) for k in range(CK)]

        @pl.when(c >= 2)
        def _():
            for k in range(CK):
                p_desc(base - 2 * CK + k, bank, k).wait()

        for k in range(CK):
            g_desc(pvs[k], bank, k).start()

        @pl.when(c >= 1)
        def _():
            for k in range(CK):
                g_desc(0, obank, k).wait()
            for k in range(CK):
                p_desc(base - CK + k, obank, k).start()

        return carry

    lax.fori_loop(0, NC, iter_c, 0)

    last = (NC - 1) % 2
    for k in range(CK):
        g_desc(0, last, k).wait()
    for k in range(CK):
        p_desc(VPW - CK + k, last, k).start()
    for k in range(CK):
        p_desc(VPW - 2 * CK + k, 1 - last, k).wait()
    for k in range(CK):
        p_desc(VPW - CK + k, last, k).wait()


def kernel(x, perm):
    perm32 = perm.astype(jnp.int32)
    return _scramble_sc(x, perm32)


# speedup vs baseline: 2.2346x; 1.7428x over previous
import functools

import jax
import jax.numpy as jnp
from jax import lax
from jax.experimental import pallas as pl
from jax.experimental.pallas import tpu as pltpu
from jax.experimental.pallas import tpu_sc as plsc

B = 32
TRACKS = 16
VARS = 512
CH = 64
NCORES = 2
CPW = CH // 2
L = 16
NJ = VARS // L


@functools.partial(
    pl.kernel,
    mesh=plsc.VectorSubcoreMesh(core_axis_name="c", subcore_axis_name="s"),
    out_type=jax.ShapeDtypeStruct((B, TRACKS, CH, VARS), jnp.float32),
    scratch_types=[
        pltpu.VMEM((VARS,), jnp.int32),
        pltpu.VMEM((2, CPW, VARS), jnp.float32),
        pltpu.VMEM((2, CPW, VARS), jnp.float32),
        pltpu.SemaphoreType.DMA,
        pltpu.SemaphoreType.DMA,
    ],
    compiler_params=pltpu.CompilerParams(needs_layout_passes=False),
)
def _scramble_sc(x_hbm, perm_hbm, out_hbm, perm_v, in_v, out_v, isem, osem):
    wid = lax.axis_index("s") * NCORES + lax.axis_index("c")
    t = wid // 2
    c0 = (wid % 2) * CPW
    pltpu.sync_copy(perm_hbm.at[t], perm_v)

    def in_desc(b, buf):
        return pltpu.make_async_copy(
            x_hbm.at[b, t, pl.ds(c0, CPW), :], in_v.at[buf], isem)

    def out_desc(b, buf):
        return pltpu.make_async_copy(
            out_v.at[buf], out_hbm.at[b, t, pl.ds(c0, CPW), :], osem)

    in_desc(0, 0).start()

    def per_batch(b, carry):
        buf = lax.rem(b, 2)

        @pl.when(b + 1 < B)
        def _():
            in_desc(b + 1, 1 - buf).start()

        in_desc(b, buf).wait()

        @pl.when(b >= 2)
        def _():
            out_desc(b - 2, buf).wait()

        src = in_v.at[buf]
        dst = out_v.at[buf]

        def per_group(j, inner):
            pvec = perm_v[pl.ds(j * L, L)]
            for c in range(CPW):
                cvec = jnp.full((L,), c, jnp.int32)
                vals = plsc.load_gather(src, [cvec, pvec])
                dst[c, pl.ds(j * L, L)] = vals
            return inner

        lax.fori_loop(0, NJ, per_group, 0)
        out_desc(b, buf).start()
        return carry

    lax.fori_loop(0, B, per_batch, 0)
    out_desc(B - 2, 0).wait()
    out_desc(B - 1, 1).wait()


def kernel(x, perm):
    perm32 = perm.astype(jnp.int32)
    xt = jnp.transpose(x, (0, 1, 3, 2))
    out_t = _scramble_sc(xt, perm32)
    return jnp.transpose(out_t, (0, 1, 3, 2))
